# baseline (speedup 1.0000x reference)
import functools

import jax
import jax.numpy as jnp
from jax import lax
from jax.experimental import pallas as pl
from jax.experimental.pallas import tpu as pltpu
from jax.experimental.pallas import tpu_sc as plsc

_NSEG = 16
_SEG = 1024
_D = 512
_L = 16
_HCOL = 256
_NG = _HCOL // _L
_CHUNK = 128
_REP = 128


def _body(x_hbm, out_hbm, in0, in1, ob, sem_in, sem_out):
    wid = lax.axis_index("c") * 16 + lax.axis_index("s")
    seg = wid // 2
    col0 = (wid % 2) * _HCOL
    in_bufs = (in0, in1)
    inv = jnp.full((_L,), 1.0 / _SEG, dtype=jnp.float32)
    n_chunks = _SEG // _CHUNK

    def in_copy(chunk, buf):
        return pltpu.make_async_copy(
            x_hbm.at[pl.ds(seg * _SEG + chunk * _CHUNK, _CHUNK),
                     pl.ds(col0, _HCOL)],
            buf, sem_in)

    in_copy(0, in_bufs[0]).start()

    accs = tuple(jnp.zeros((_L,), jnp.float32) for _ in range(_NG))
    for chunk in range(n_chunks):
        buf = in_bufs[chunk % 2]
        in_copy(chunk, buf).wait()
        if chunk + 1 < n_chunks:
            in_copy(chunk + 1, in_bufs[(chunk + 1) % 2]).start()

        def red_step(t, a):
            r0 = t * 8
            for r in range(8):
                a = tuple(
                    a[g] + buf[r0 + r, pl.ds(g * _L, _L)]
                    for g in range(_NG)
                )
            return a

        accs = lax.fori_loop(0, _CHUNK // 8, red_step, accs)

    means = tuple(a * inv for a in accs)

    def rep_step(i, _):
        for g in range(_NG):
            ob[i, pl.ds(g * _L, _L)] = means[g]
        return 0

    lax.fori_loop(0, _REP, rep_step, 0)

    for r in range(_SEG // _REP):
        pltpu.make_async_copy(
            ob,
            out_hbm.at[pl.ds(seg * _SEG + r * _REP, _REP),
                       pl.ds(col0, _HCOL)],
            sem_out).start()

    for _ in range(_SEG // _REP):
        pltpu.make_async_copy(
            ob, out_hbm.at[pl.ds(0, _REP), pl.ds(0, _HCOL)], sem_out
        ).wait()


def kernel(embedded_site_features):
    mesh = plsc.VectorSubcoreMesh(core_axis_name="c", subcore_axis_name="s")
    total = _NSEG * _SEG
    run = functools.partial(
        pl.kernel,
        mesh=mesh,
        out_type=jax.ShapeDtypeStruct((total, _D), jnp.float32),
        scratch_types=[
            pltpu.VMEM((_CHUNK, _HCOL), jnp.float32),
            pltpu.VMEM((_CHUNK, _HCOL), jnp.float32),
            pltpu.VMEM((_REP, _HCOL), jnp.float32),
            pltpu.SemaphoreType.DMA,
            pltpu.SemaphoreType.DMA,
        ],
        compiler_params=pltpu.CompilerParams(use_tc_tiling_on_sc=True),
    )(_body)
    return run(embedded_site_features)
